# baseline (speedup 1.0000x reference)
import functools

import jax
import jax.numpy as jnp
from jax import lax
from jax.experimental import pallas as pl
from jax.experimental.pallas import tpu as pltpu
from jax.experimental.pallas import tpu_sc as plsc

_INFO = plsc.get_sparse_core_info()
_NC = _INFO.num_cores
_NS = _INFO.num_subcores
_NW = _NC * _NS

_SC_ROWS = 8192
_SC_CHUNK = 128
_TC_CHUNK = 1024


def _sc_program(rows, w, sc_base):
    rpw = _SC_ROWS // _NW
    nchunks = rpw // _SC_CHUNK
    mesh = plsc.VectorSubcoreMesh(core_axis_name="c", subcore_axis_name="s")

    @functools.partial(
        pl.kernel,
        out_type=jax.ShapeDtypeStruct((rows, w), jnp.float32),
        mesh=mesh,
        scratch_types=[
            pltpu.VMEM_SHARED((2, _NS * _SC_CHUNK, w), jnp.float32),
            pltpu.SemaphoreType.DMA((2,)),
            pltpu.SemaphoreType.DMA((2,)),
        ],
    )
    def _sc_slot_scatter(src_hbm, out_hbm, buf, in_sems, out_sems):
        sid = lax.axis_index("s")
        wid = sid * _NC + lax.axis_index("c")
        base = sc_base + wid * rpw

        def in_copy(k):
            return pltpu.make_async_copy(
                src_hbm.at[pl.ds(base + k * _SC_CHUNK, _SC_CHUNK)],
                buf.at[k % 2, pl.ds(sid * _SC_CHUNK, _SC_CHUNK)],
                in_sems.at[k % 2],
            )

        def out_copy(k):
            return pltpu.make_async_copy(
                buf.at[k % 2, pl.ds(sid * _SC_CHUNK, _SC_CHUNK)],
                out_hbm.at[pl.ds(base + k * _SC_CHUNK, _SC_CHUNK)],
                out_sems.at[k % 2],
            )

        in_copy(0).start()
        for k in range(nchunks):
            in_copy(k).wait()
            if k + 1 < nchunks:
                if k >= 1:
                    out_copy(k - 1).wait()
                in_copy(k + 1).start()
            out_copy(k).start()
        out_copy(nchunks - 2).wait()
        out_copy(nchunks - 1).wait()

    return _sc_slot_scatter


def _tc_head_copy(src_hbm, partial_hbm, dst_hbm, buf, in_sems, out_sems):
    del partial_hbm
    tc_rows = src_hbm.shape[0] - _SC_ROWS
    nchunks = tc_rows // _TC_CHUNK
    for i in range(nchunks):
        sl = pl.ds(i * _TC_CHUNK, _TC_CHUNK)
        pltpu.make_async_copy(src_hbm.at[sl], buf.at[sl], in_sems.at[i]).start()
    for i in range(nchunks):
        sl = pl.ds(i * _TC_CHUNK, _TC_CHUNK)
        pltpu.make_async_copy(src_hbm.at[sl], buf.at[sl], in_sems.at[i]).wait()
        pltpu.make_async_copy(buf.at[sl], dst_hbm.at[sl], out_sems.at[i]).start()
    for i in range(nchunks):
        sl = pl.ds(i * _TC_CHUNK, _TC_CHUNK)
        pltpu.make_async_copy(buf.at[sl], dst_hbm.at[sl], out_sems.at[i]).wait()


def kernel(img_batch, canvases):
    del canvases
    b, c, h, w = img_batch.shape
    rows = b * c * h
    flat = img_batch.reshape(rows, w)
    tc_rows = rows - _SC_ROWS

    partial = _sc_program(rows, w, tc_rows)(flat)

    n_tc_chunks = tc_rows // _TC_CHUNK
    out = pl.pallas_call(
        _tc_head_copy,
        in_specs=[
            pl.BlockSpec(memory_space=pltpu.MemorySpace.HBM),
            pl.BlockSpec(memory_space=pltpu.MemorySpace.HBM),
        ],
        out_specs=pl.BlockSpec(memory_space=pltpu.MemorySpace.HBM),
        out_shape=jax.ShapeDtypeStruct(flat.shape, flat.dtype),
        scratch_shapes=[
            pltpu.VMEM((tc_rows, w), jnp.float32),
            pltpu.SemaphoreType.DMA((n_tc_chunks,)),
            pltpu.SemaphoreType.DMA((n_tc_chunks,)),
        ],
        input_output_aliases={1: 0},
    )(flat, partial)
    return out.reshape(b, c, h, w)


# speedup vs baseline: 1.0140x; 1.0140x over previous
import functools

import jax
import jax.numpy as jnp
from jax import lax
from jax.experimental import pallas as pl
from jax.experimental.pallas import tpu as pltpu
from jax.experimental.pallas import tpu_sc as plsc

_INFO = plsc.get_sparse_core_info()
_NC = _INFO.num_cores
_NS = _INFO.num_subcores
_NW = _NC * _NS

_SC_ROWS = 8192
_SC_CHUNK = 128
_TC_BLOCK = 4096


def _sc_program(rows, w, sc_base):
    rpw = _SC_ROWS // _NW
    nchunks = rpw // _SC_CHUNK
    mesh = plsc.VectorSubcoreMesh(core_axis_name="c", subcore_axis_name="s")

    @functools.partial(
        pl.kernel,
        out_type=jax.ShapeDtypeStruct((rows, w), jnp.float32),
        mesh=mesh,
        scratch_types=[
            pltpu.VMEM_SHARED((2, _NS * _SC_CHUNK, w), jnp.float32),
            pltpu.SemaphoreType.DMA((2,)),
            pltpu.SemaphoreType.DMA((2,)),
        ],
    )
    def _sc_slot_scatter(src_hbm, out_hbm, buf, in_sems, out_sems):
        sid = lax.axis_index("s")
        wid = sid * _NC + lax.axis_index("c")
        base = sc_base + wid * rpw

        def in_copy(k):
            return pltpu.make_async_copy(
                src_hbm.at[pl.ds(base + k * _SC_CHUNK, _SC_CHUNK)],
                buf.at[k % 2, pl.ds(sid * _SC_CHUNK, _SC_CHUNK)],
                in_sems.at[k % 2],
            )

        def out_copy(k):
            return pltpu.make_async_copy(
                buf.at[k % 2, pl.ds(sid * _SC_CHUNK, _SC_CHUNK)],
                out_hbm.at[pl.ds(base + k * _SC_CHUNK, _SC_CHUNK)],
                out_sems.at[k % 2],
            )

        in_copy(0).start()
        for k in range(nchunks):
            in_copy(k).wait()
            if k + 1 < nchunks:
                if k >= 1:
                    out_copy(k - 1).wait()
                in_copy(k + 1).start()
            out_copy(k).start()
        out_copy(nchunks - 2).wait()
        out_copy(nchunks - 1).wait()

    return _sc_slot_scatter


def _tc_head_copy(src_ref, partial_hbm, dst_ref):
    del partial_hbm
    dst_ref[...] = src_ref[...]


def kernel(img_batch, canvases):
    del canvases
    b, c, h, w = img_batch.shape
    rows = b * c * h
    flat = img_batch.reshape(rows, w)
    tc_rows = rows - _SC_ROWS

    partial = _sc_program(rows, w, tc_rows)(flat)

    grid = tc_rows // _TC_BLOCK
    out = pl.pallas_call(
        _tc_head_copy,
        grid=(grid,),
        in_specs=[
            pl.BlockSpec((_TC_BLOCK, w), lambda i: (i, 0)),
            pl.BlockSpec(memory_space=pltpu.MemorySpace.HBM),
        ],
        out_specs=pl.BlockSpec((_TC_BLOCK, w), lambda i: (i, 0)),
        out_shape=jax.ShapeDtypeStruct(flat.shape, flat.dtype),
        input_output_aliases={1: 0},
    )(flat, partial)
    return out.reshape(b, c, h, w)


# speedup vs baseline: 1.0401x; 1.0257x over previous
import functools

import jax
import jax.numpy as jnp
from jax import lax
from jax.experimental import pallas as pl
from jax.experimental.pallas import tpu as pltpu
from jax.experimental.pallas import tpu_sc as plsc

_INFO = plsc.get_sparse_core_info()
_NC = _INFO.num_cores
_NS = _INFO.num_subcores
_NW = _NC * _NS

_SC_ROWS = 4096
_SC_CHUNK = 64
_TC_BLOCK = 4096


def _sc_program(rows, w, sc_base):
    rpw = _SC_ROWS // _NW
    nchunks = rpw // _SC_CHUNK
    mesh = plsc.VectorSubcoreMesh(core_axis_name="c", subcore_axis_name="s")

    @functools.partial(
        pl.kernel,
        out_type=jax.ShapeDtypeStruct((rows, w), jnp.float32),
        mesh=mesh,
        scratch_types=[
            pltpu.VMEM_SHARED((2, _NS * _SC_CHUNK, w), jnp.float32),
            pltpu.SemaphoreType.DMA((2,)),
            pltpu.SemaphoreType.DMA((2,)),
        ],
    )
    def _sc_slot_scatter(src_hbm, out_hbm, buf, in_sems, out_sems):
        sid = lax.axis_index("s")
        wid = sid * _NC + lax.axis_index("c")
        base = sc_base + wid * rpw

        def in_copy(k):
            return pltpu.make_async_copy(
                src_hbm.at[pl.ds(base + k * _SC_CHUNK, _SC_CHUNK)],
                buf.at[k % 2, pl.ds(sid * _SC_CHUNK, _SC_CHUNK)],
                in_sems.at[k % 2],
            )

        def out_copy(k):
            return pltpu.make_async_copy(
                buf.at[k % 2, pl.ds(sid * _SC_CHUNK, _SC_CHUNK)],
                out_hbm.at[pl.ds(base + k * _SC_CHUNK, _SC_CHUNK)],
                out_sems.at[k % 2],
            )

        in_copy(0).start()
        for k in range(nchunks):
            in_copy(k).wait()
            if k + 1 < nchunks:
                if k >= 1:
                    out_copy(k - 1).wait()
                in_copy(k + 1).start()
            out_copy(k).start()
        out_copy(nchunks - 2).wait()
        out_copy(nchunks - 1).wait()

    return _sc_slot_scatter


def _tc_head_copy(src_ref, partial_hbm, dst_ref):
    del partial_hbm
    dst_ref[...] = src_ref[...]


def kernel(img_batch, canvases):
    del canvases
    b, c, h, w = img_batch.shape
    rows = b * c * h
    flat = img_batch.reshape(rows, w)
    tc_rows = rows - _SC_ROWS

    partial = _sc_program(rows, w, tc_rows)(flat)

    grid = tc_rows // _TC_BLOCK
    out = pl.pallas_call(
        _tc_head_copy,
        grid=(grid,),
        in_specs=[
            pl.BlockSpec((_TC_BLOCK, w), lambda i: (i, 0)),
            pl.BlockSpec(memory_space=pltpu.MemorySpace.HBM),
        ],
        out_specs=pl.BlockSpec((_TC_BLOCK, w), lambda i: (i, 0)),
        out_shape=jax.ShapeDtypeStruct(flat.shape, flat.dtype),
        input_output_aliases={1: 0},
    )(flat, partial)
    return out.reshape(b, c, h, w)


# speedup vs baseline: 1.0503x; 1.0098x over previous
import functools

import jax
import jax.numpy as jnp
from jax import lax
from jax.experimental import pallas as pl
from jax.experimental.pallas import tpu as pltpu
from jax.experimental.pallas import tpu_sc as plsc

_INFO = plsc.get_sparse_core_info()
_NC = _INFO.num_cores
_NS = _INFO.num_subcores
_NW = _NC * _NS

_SC_ROWS = 4096
_SC_CHUNK = 64
_TC_BLOCK = 5120


def _sc_program(rows, w, sc_base):
    rpw = _SC_ROWS // _NW
    nchunks = rpw // _SC_CHUNK
    mesh = plsc.VectorSubcoreMesh(core_axis_name="c", subcore_axis_name="s")

    @functools.partial(
        pl.kernel,
        out_type=jax.ShapeDtypeStruct((rows, w), jnp.float32),
        mesh=mesh,
        scratch_types=[
            pltpu.VMEM_SHARED((2, _NS * _SC_CHUNK, w), jnp.float32),
            pltpu.SemaphoreType.DMA((2,)),
            pltpu.SemaphoreType.DMA((2,)),
        ],
    )
    def _sc_slot_scatter(src_hbm, out_hbm, buf, in_sems, out_sems):
        sid = lax.axis_index("s")
        wid = sid * _NC + lax.axis_index("c")
        base = sc_base + wid * rpw

        def in_copy(k):
            return pltpu.make_async_copy(
                src_hbm.at[pl.ds(base + k * _SC_CHUNK, _SC_CHUNK)],
                buf.at[k % 2, pl.ds(sid * _SC_CHUNK, _SC_CHUNK)],
                in_sems.at[k % 2],
            )

        def out_copy(k):
            return pltpu.make_async_copy(
                buf.at[k % 2, pl.ds(sid * _SC_CHUNK, _SC_CHUNK)],
                out_hbm.at[pl.ds(base + k * _SC_CHUNK, _SC_CHUNK)],
                out_sems.at[k % 2],
            )

        in_copy(0).start()
        for k in range(nchunks):
            in_copy(k).wait()
            if k + 1 < nchunks:
                if k >= 1:
                    out_copy(k - 1).wait()
                in_copy(k + 1).start()
            out_copy(k).start()
        out_copy(nchunks - 2).wait()
        out_copy(nchunks - 1).wait()

    return _sc_slot_scatter


def _tc_head_copy(src_ref, partial_hbm, dst_ref):
    del partial_hbm
    dst_ref[...] = src_ref[...]


def kernel(img_batch, canvases):
    del canvases
    b, c, h, w = img_batch.shape
    rows = b * c * h
    flat = img_batch.reshape(rows, w)
    tc_rows = rows - _SC_ROWS

    partial = _sc_program(rows, w, tc_rows)(flat)

    grid = tc_rows // _TC_BLOCK
    out = pl.pallas_call(
        _tc_head_copy,
        grid=(grid,),
        in_specs=[
            pl.BlockSpec((_TC_BLOCK, w), lambda i: (i, 0)),
            pl.BlockSpec(memory_space=pltpu.MemorySpace.HBM),
        ],
        out_specs=pl.BlockSpec((_TC_BLOCK, w), lambda i: (i, 0)),
        out_shape=jax.ShapeDtypeStruct(flat.shape, flat.dtype),
        input_output_aliases={1: 0},
    )(flat, partial)
    return out.reshape(b, c, h, w)
